# baseline (speedup 1.0000x reference)
import functools

import jax
import jax.numpy as jnp
from jax import lax
from jax.experimental import pallas as pl
from jax.experimental.pallas import tpu as pltpu
from jax.experimental.pallas import tpu_sc as plsc

_NUM_CORES = 2
_NUM_SUBCORES = 16
_LANES = 16


def _sc_dots(centerWords, contexts, inputEmbedding, outputEmbedding):
    B, K = contexts.shape
    E = inputEmbedding.shape[1]
    L = _LANES
    NW = _NUM_CORES * _NUM_SUBCORES
    BW = B // NW
    EC = E // L
    KP = ((K + E - 1) // E) * E

    mesh = plsc.VectorSubcoreMesh(core_axis_name="c", subcore_axis_name="s")

    @functools.partial(
        pl.kernel,
        out_type=jax.ShapeDtypeStruct((B, KP), jnp.float32),
        mesh=mesh,
        compiler_params=pltpu.CompilerParams(
            needs_layout_passes=False, use_tc_tiling_on_sc=False),
        scratch_types=[
            pltpu.VMEM((BW,), jnp.int32),
            pltpu.VMEM((BW, K), jnp.int32),
            pltpu.VMEM((BW, E), jnp.float32),
            pltpu.VMEM((KP, E // 2), jnp.int32),
            pltpu.VMEM((KP, E // 2), jnp.int32),
            pltpu.VMEM((L, L), jnp.float32),
            pltpu.VMEM((L, L), jnp.float32),
            pltpu.VMEM((BW, KP), jnp.float32),
            pltpu.SemaphoreType.DMA,
            pltpu.SemaphoreType.DMA,
            pltpu.SemaphoreType.DMA,
        ],
    )
    def sc_kernel(cw_hbm, ctx_hbm, in_emb, out_emb, out_hbm,
                  cidx_v, xidx_v, c_v, rows0_v, rows1_v, t0_v, t1_v, dots_v,
                  sem_c, sem0, sem1):
        wid = lax.axis_index("s") * _NUM_CORES + lax.axis_index("c")
        base = wid * BW
        pltpu.sync_copy(cw_hbm.at[pl.ds(base, BW)], cidx_v)
        pltpu.sync_copy(ctx_hbm.at[pl.ds(base, BW), :], xidx_v)
        pltpu.async_copy(in_emb.at[cidx_v], c_v, sem_c).wait()

        zeros_i = jnp.zeros((L,), jnp.int32)
        for rows_v in (rows0_v, rows1_v):
            for r in range(K, KP):
                for j in range(E // (2 * L)):
                    rows_v[r, pl.ds(j * L, L)] = zeros_i

        lane_iota = lax.broadcasted_iota(jnp.int32, (L,), 0)

        def start(b, rows_v, sem):
            pltpu.async_copy(out_emb.at[xidx_v.at[b]], rows_v.at[pl.ds(0, K)], sem)

        def wait(rows_v, sem):
            pltpu.make_async_copy(
                out_emb.at[pl.ds(0, K)], rows_v.at[pl.ds(0, K)], sem).wait()

        diag = [jnp.bitwise_and(lane_iota + l, L - 1) for l in range(L)]
        rowc = [jnp.full((L,), l, jnp.int32) for l in range(L)]

        def compute(b, rows_v):
            bvec = jnp.full((L,), b, jnp.int32)
            cpk = []
            for j in range(E // (2 * L)):
                ce = plsc.load_gather(c_v, [bvec, lane_iota * 2 + j * 2 * L])
                co = plsc.load_gather(c_v, [bvec, lane_iota * 2 + 1 + j * 2 * L])
                cpk.append(plsc.pack(ce, co, format=plsc.PackFormat.INTERLEAVED))

            @pl.loop(0, KP, step=2 * L)
            def _grp2(k00):
                for t_v, k0 in ((t0_v, k00), (t1_v, k00 + L)):
                    for l in range(L):
                        prods = [
                            plsc.bitcast(rows_v[k0 + l, pl.ds(j * L, L)],
                                         jnp.bfloat16) * cpk[j]
                            for j in range(E // (2 * L))
                        ]
                        while len(prods) > 1:
                            prods = [x + y for x, y in zip(prods[::2], prods[1::2])]
                        pa, pb = plsc.unpack(
                            prods[0], format=plsc.PackFormat.INTERLEAVED)
                        plsc.store_scatter(t_v, [rowc[l], diag[l]], pa + pb)
                    cols = [
                        plsc.load_gather(t_v, [lane_iota, diag[s]])
                        for s in range(L)
                    ]
                    while len(cols) > 1:
                        cols = [a + b2 for a, b2 in zip(cols[::2], cols[1::2])]
                    dots_v[b, pl.ds(k0, L)] = cols[0]

        start(0, rows0_v, sem0)

        @pl.loop(0, BW, step=2)
        def _pair(b):
            start(b + 1, rows1_v, sem1)
            wait(rows0_v, sem0)
            compute(b, rows0_v)

            @pl.when(b + 2 < BW)
            def _():
                start(b + 2, rows0_v, sem0)

            wait(rows1_v, sem1)
            compute(b + 1, rows1_v)

        pltpu.sync_copy(dots_v, out_hbm.at[pl.ds(base, BW), :])

    return sc_kernel(centerWords, contexts, inputEmbedding, outputEmbedding)


def _tc_loss(dots, num_pos, num_valid):
    B, KP = dots.shape

    def body(d_ref, o_ref):
        d = d_ref[...]
        col = lax.broadcasted_iota(jnp.int32, (B, KP), 1)
        x = jnp.where(col < num_pos, d, -d)
        ls = jnp.minimum(x, 0.0) - jnp.log1p(jnp.exp(-jnp.abs(x)))
        ls = jnp.where(col < num_valid, ls, 0.0)
        o_ref[0, 0] = -jnp.sum(ls) / B

    return pl.pallas_call(
        body,
        out_shape=jax.ShapeDtypeStruct((1, 1), jnp.float32),
        out_specs=pl.BlockSpec(memory_space=pltpu.SMEM),
    )(dots)


def kernel(centerWords, positiveWords, negativeWords, inputEmbedding, outputEmbedding):
    P = positiveWords.shape[1]
    contexts = jnp.concatenate(
        [positiveWords.astype(jnp.int32), negativeWords.astype(jnp.int32)], axis=1)
    K = contexts.shape[1]
    V, E = outputEmbedding.shape
    ctx_table = jax.lax.bitcast_convert_type(
        outputEmbedding.astype(jnp.bfloat16).reshape(V, E // 2, 2), jnp.int32)
    dots = _sc_dots(centerWords.astype(jnp.int32), contexts,
                    inputEmbedding, ctx_table)
    loss = _tc_loss(dots, P, K)
    return jnp.reshape(loss, ())


# speedup vs baseline: 3.0566x; 3.0566x over previous
import functools

import jax
import jax.numpy as jnp
from jax import lax
from jax.experimental import pallas as pl
from jax.experimental.pallas import tpu as pltpu
from jax.experimental.pallas import tpu_sc as plsc

_NUM_CORES = 2
_NUM_SUBCORES = 16
_LANES = 16


def _sc_dots(centerWords, contexts, inputEmbedding, outputEmbedding):
    B, K = contexts.shape
    E = inputEmbedding.shape[1]
    L = _LANES
    NW = _NUM_CORES * _NUM_SUBCORES
    BW = B // NW
    EC = E // L
    KP = ((K + E - 1) // E) * E

    mesh = plsc.VectorSubcoreMesh(core_axis_name="c", subcore_axis_name="s")

    @functools.partial(
        pl.kernel,
        out_type=jax.ShapeDtypeStruct((B, KP), jnp.float32),
        mesh=mesh,
        compiler_params=pltpu.CompilerParams(needs_layout_passes=False),
        scratch_types=[
            pltpu.VMEM((BW,), jnp.int32),
            pltpu.VMEM((BW, K), jnp.int32),
            pltpu.VMEM((BW, E), jnp.float32),
            pltpu.VMEM((KP, E), jnp.float32),
            pltpu.VMEM((KP, E), jnp.float32),
            pltpu.VMEM((L, L), jnp.float32),
            pltpu.VMEM((L, L), jnp.float32),
            pltpu.VMEM((BW, KP), jnp.float32),
            pltpu.SemaphoreType.DMA,
            pltpu.SemaphoreType.DMA,
            pltpu.SemaphoreType.DMA,
        ],
    )
    def sc_kernel(cw_hbm, ctx_hbm, in_emb, out_emb, out_hbm,
                  cidx_v, xidx_v, c_v, rows0_v, rows1_v, t0_v, t1_v, dots_v,
                  sem_c, sem0, sem1):
        wid = lax.axis_index("s") * _NUM_CORES + lax.axis_index("c")
        base = wid * BW
        pltpu.sync_copy(cw_hbm.at[pl.ds(base, BW)], cidx_v)
        pltpu.sync_copy(ctx_hbm.at[pl.ds(base, BW), :], xidx_v)
        pltpu.async_copy(in_emb.at[cidx_v], c_v, sem_c).wait()

        zeros = jnp.zeros((L,), jnp.float32)
        for rows_v in (rows0_v, rows1_v):
            for r in range(K, KP):
                for j in range(EC):
                    rows_v[r, pl.ds(j * L, L)] = zeros

        lane_iota = lax.broadcasted_iota(jnp.int32, (L,), 0)

        def start(b, rows_v, sem):
            pltpu.async_copy(out_emb.at[xidx_v.at[b]], rows_v.at[pl.ds(0, K)], sem)

        def wait(rows_v, sem):
            pltpu.make_async_copy(
                out_emb.at[pl.ds(0, K)], rows_v.at[pl.ds(0, K)], sem).wait()

        diag = [jnp.bitwise_and(lane_iota + l, L - 1) for l in range(L)]
        rowc = [jnp.full((L,), l, jnp.int32) for l in range(L)]

        def compute(b, rows_v):
            cvecs = [c_v[b, pl.ds(j * L, L)] for j in range(EC)]

            @pl.loop(0, KP, step=2 * L)
            def _grp2(k00):
                for t_v, k0 in ((t0_v, k00), (t1_v, k00 + L)):
                    for l in range(L):
                        acc0 = rows_v[k0 + l, pl.ds(0, L)] * cvecs[0]
                        acc1 = rows_v[k0 + l, pl.ds(L, L)] * cvecs[1]
                        for j in range(2, EC, 2):
                            acc0 = acc0 + rows_v[k0 + l, pl.ds(j * L, L)] * cvecs[j]
                            acc1 = acc1 + rows_v[k0 + l, pl.ds((j + 1) * L, L)] * cvecs[j + 1]
                        plsc.store_scatter(t_v, [rowc[l], diag[l]], acc0 + acc1)
                    cols = [
                        plsc.load_gather(t_v, [lane_iota, diag[s]])
                        for s in range(L)
                    ]
                    while len(cols) > 1:
                        cols = [a + b2 for a, b2 in zip(cols[::2], cols[1::2])]
                    dots_v[b, pl.ds(k0, L)] = cols[0]

        start(0, rows0_v, sem0)

        @pl.loop(0, BW, step=2)
        def _pair(b):
            start(b + 1, rows1_v, sem1)
            wait(rows0_v, sem0)
            compute(b, rows0_v)

            @pl.when(b + 2 < BW)
            def _():
                start(b + 2, rows0_v, sem0)

            wait(rows1_v, sem1)
            compute(b + 1, rows1_v)

        pltpu.sync_copy(dots_v, out_hbm.at[pl.ds(base, BW), :])

    return sc_kernel(centerWords, contexts, inputEmbedding, outputEmbedding)


def _tc_loss(dots, num_pos, num_valid):
    B, KP = dots.shape

    def body(d_ref, o_ref):
        d = d_ref[...]
        col = lax.broadcasted_iota(jnp.int32, (B, KP), 1)
        x = jnp.where(col < num_pos, d, -d)
        ls = jnp.minimum(x, 0.0) - jnp.log1p(jnp.exp(-jnp.abs(x)))
        ls = jnp.where(col < num_valid, ls, 0.0)
        o_ref[0, 0] = -jnp.sum(ls) / B

    return pl.pallas_call(
        body,
        out_shape=jax.ShapeDtypeStruct((1, 1), jnp.float32),
        out_specs=pl.BlockSpec(memory_space=pltpu.SMEM),
    )(dots)


def kernel(centerWords, positiveWords, negativeWords, inputEmbedding, outputEmbedding):
    P = positiveWords.shape[1]
    contexts = jnp.concatenate(
        [positiveWords.astype(jnp.int32), negativeWords.astype(jnp.int32)], axis=1)
    K = contexts.shape[1]
    dots = _sc_dots(centerWords.astype(jnp.int32), contexts,
                    inputEmbedding, outputEmbedding)
    loss = _tc_loss(dots, P, K)
    return jnp.reshape(loss, ())
